# baseline (speedup 1.0000x reference)
import jax
import jax.numpy as jnp
from jax.experimental import pallas as pl
from jax.experimental.pallas import tpu as pltpu

_BM = 200


def _agg_kernel(a_ref, x_ref, ego_ref, wt_ref, b_ref, out_ref):
    s = jnp.dot(a_ref[...], x_ref[...], preferred_element_type=jnp.float32)
    y = ego_ref[...] + s
    y = jnp.dot(y, wt_ref[...], preferred_element_type=jnp.float32)
    y = y + b_ref[...]
    out_ref[...] = jnp.where(y >= 0.0, y, 0.01 * y)


def kernel(ego_embeddings, A_in, W, b):
    N, D = ego_embeddings.shape
    nm = N // _BM
    wt = W.T
    b2 = b.reshape(1, D)

    return pl.pallas_call(
        _agg_kernel,
        grid=(nm,),
        in_specs=[
            pl.BlockSpec((_BM, N), lambda i: (i, 0)),
            pl.BlockSpec((N, D), lambda i: (0, 0)),
            pl.BlockSpec((_BM, D), lambda i: (i, 0)),
            pl.BlockSpec((D, D), lambda i: (0, 0)),
            pl.BlockSpec((1, D), lambda i: (0, 0)),
        ],
        out_specs=pl.BlockSpec((_BM, D), lambda i: (i, 0)),
        out_shape=jax.ShapeDtypeStruct((N, D), jnp.float32),
        compiler_params=pltpu.CompilerParams(
            dimension_semantics=("arbitrary",),
        ),
    )(A_in, ego_embeddings, ego_embeddings, wt, b2)


# speedup vs baseline: 1.0273x; 1.0273x over previous
import jax
import jax.numpy as jnp
from jax.experimental import pallas as pl
from jax.experimental.pallas import tpu as pltpu

_BM = 200


def _agg_kernel(a0_ref, a1_ref, x_ref, ego_ref, wt_ref, b_ref, out_ref):
    s0 = jnp.dot(a0_ref[...], x_ref[...], preferred_element_type=jnp.float32)
    s1 = jnp.dot(a1_ref[...], x_ref[...], preferred_element_type=jnp.float32)
    y = ego_ref[...] + jnp.concatenate([s0, s1], axis=0)
    y = jnp.dot(y, wt_ref[...], preferred_element_type=jnp.float32)
    y = y + b_ref[...]
    out_ref[...] = jnp.where(y >= 0.0, y, 0.01 * y)


def kernel(ego_embeddings, A_in, W, b):
    N, D = ego_embeddings.shape
    nm = N // (2 * _BM)
    wt = W.T
    b2 = b.reshape(1, D)

    return pl.pallas_call(
        _agg_kernel,
        grid=(nm,),
        in_specs=[
            pl.BlockSpec((_BM, N), lambda i: (2 * i, 0)),
            pl.BlockSpec((_BM, N), lambda i: (2 * i + 1, 0)),
            pl.BlockSpec((N, D), lambda i: (0, 0)),
            pl.BlockSpec((2 * _BM, D), lambda i: (i, 0)),
            pl.BlockSpec((D, D), lambda i: (0, 0)),
            pl.BlockSpec((1, D), lambda i: (0, 0)),
        ],
        out_specs=pl.BlockSpec((2 * _BM, D), lambda i: (i, 0)),
        out_shape=jax.ShapeDtypeStruct((N, D), jnp.float32),
        compiler_params=pltpu.CompilerParams(
            dimension_semantics=("arbitrary",),
        ),
    )(A_in, A_in, ego_embeddings, ego_embeddings, wt, b2)
